# baseline (device time: 190802 ns/iter reference)
import jax
import jax.numpy as jnp
from jax import lax
from jax.experimental import pallas as pl
from jax.experimental.pallas import tpu as pltpu

N_DEV = 4
W = 16


def kernel(table, idx):
    v_per, d = table.shape
    n = idx.shape[0]
    c = n // N_DEV

    idx2d = idx[:, None]

    def body(table_ref, idx_ref, idx2_ref, out_ref,
             gbuf_ref, comm_ref, gather_sems, send_sems, recv_sems):
        me = lax.axis_index("i")
        left = lax.rem(me + N_DEV - 1, N_DEV)
        right = lax.rem(me + 1, N_DEV)
        lo = me * v_per

        def row_copy(i):
            g = idx_ref[i]
            l = jnp.clip(g - lo, 0, v_per - 1)
            return pltpu.make_async_copy(
                table_ref.at[pl.ds(l, 1)],
                gbuf_ref.at[pl.ds(i, 1)],
                gather_sems.at[lax.rem(i, W)],
            )

        def gather_step(i, _):
            @pl.when(i >= W)
            def _():
                row_copy(i - W).wait()
            row_copy(i).start()
            return 0

        lax.fori_loop(0, n, gather_step, 0)

        def drain_step(k, _):
            row_copy(n - W + k).wait()
            return 0

        lax.fori_loop(0, W, drain_step, 0)

        owned = (idx2_ref[...] >= lo) & (idx2_ref[...] < lo + v_per)
        out_ref[...] = jnp.where(
            owned, gbuf_ref[...], 0.0
        ).astype(jnp.bfloat16)

        barrier_sem = pltpu.get_barrier_semaphore()
        for nbr in (left, right):
            pl.semaphore_signal(
                barrier_sem, inc=1,
                device_id=(nbr,), device_id_type=pl.DeviceIdType.MESH,
            )
        pl.semaphore_wait(barrier_sem, 2)

        for s in range(N_DEV - 1):
            send_chunk = lax.rem(me - s + N_DEV, N_DEV)
            recv_chunk = lax.rem(me - s - 1 + N_DEV, N_DEV)
            rdma = pltpu.make_async_remote_copy(
                src_ref=out_ref.at[pl.ds(send_chunk * c, c)],
                dst_ref=comm_ref.at[s],
                send_sem=send_sems.at[s],
                recv_sem=recv_sems.at[s],
                device_id=(right,),
                device_id_type=pl.DeviceIdType.MESH,
            )
            rdma.start()
            rdma.wait()
            off = recv_chunk * c
            out_ref[pl.ds(off, c), :] = out_ref[pl.ds(off, c), :] + comm_ref[s]

        for s in range(N_DEV - 1):
            chunk = lax.rem(me + 1 - s + N_DEV, N_DEV)
            off = chunk * c
            rdma = pltpu.make_async_remote_copy(
                src_ref=out_ref.at[pl.ds(off, c)],
                dst_ref=out_ref.at[pl.ds(off, c)],
                send_sem=send_sems.at[N_DEV - 1 + s],
                recv_sem=recv_sems.at[N_DEV - 1 + s],
                device_id=(right,),
                device_id_type=pl.DeviceIdType.MESH,
            )
            rdma.start()
            rdma.wait()

    n_steps = 2 * (N_DEV - 1)
    return pl.pallas_call(
        body,
        out_shape=jax.ShapeDtypeStruct((n, d), jnp.bfloat16),
        in_specs=[
            pl.BlockSpec(memory_space=pl.ANY),
            pl.BlockSpec(memory_space=pltpu.SMEM),
            pl.BlockSpec(memory_space=pltpu.VMEM),
        ],
        out_specs=pl.BlockSpec(memory_space=pltpu.VMEM),
        scratch_shapes=[
            pltpu.VMEM((n, d), jnp.float32),
            pltpu.VMEM((N_DEV - 1, c, d), jnp.bfloat16),
            pltpu.SemaphoreType.DMA((W,)),
            pltpu.SemaphoreType.DMA((n_steps,)),
            pltpu.SemaphoreType.DMA((n_steps,)),
        ],
        compiler_params=pltpu.CompilerParams(collective_id=0),
    )(table, idx, idx2d)


# device time: 131647 ns/iter; 1.4493x vs baseline; 1.4493x over previous
import jax
import jax.numpy as jnp
from jax import lax
from jax.experimental import pallas as pl
from jax.experimental.pallas import tpu as pltpu

N_DEV = 4
W = 16


def kernel(table, idx):
    v_per, d = table.shape
    n = idx.shape[0]
    c = n // N_DEV

    idx2d = idx[:, None]

    def body(table_ref, idx_ref, idx2_ref, out_ref,
             gbuf_ref, comm_ref, gather_sems, send_sems, recv_sems):
        me = lax.axis_index("i")
        left = lax.rem(me + N_DEV - 1, N_DEV)
        right = lax.rem(me + 1, N_DEV)
        lo = me * v_per

        def owned_scalar(i):
            g = idx_ref[i]
            return (g >= lo) & (g < lo + v_per)

        def row_copy(i):
            g = idx_ref[i]
            l = jnp.clip(g - lo, 0, v_per - 1)
            return pltpu.make_async_copy(
                table_ref.at[pl.ds(l, 1)],
                gbuf_ref.at[pl.ds(i, 1)],
                gather_sems.at[lax.rem(i, W)],
            )

        def gather_chunk(chunk):
            off = chunk * c

            def step(j, _):
                @pl.when(j >= W)
                def _():
                    jw = off + j - W

                    @pl.when(owned_scalar(jw))
                    def _():
                        row_copy(jw).wait()

                @pl.when(owned_scalar(off + j))
                def _():
                    row_copy(off + j).start()
                return 0

            lax.fori_loop(0, c, step, 0)

            def drain(k, _):
                @pl.when(owned_scalar(off + c - W + k))
                def _():
                    row_copy(off + c - W + k).wait()
                return 0

            lax.fori_loop(0, W, drain, 0)

            owned = (idx2_ref[pl.ds(off, c), :] >= lo) & (
                idx2_ref[pl.ds(off, c), :] < lo + v_per
            )
            out_ref[pl.ds(off, c), :] = jnp.where(
                owned, gbuf_ref[pl.ds(off, c), :], 0.0
            ).astype(jnp.bfloat16)

        gather_chunk(me)

        barrier_sem = pltpu.get_barrier_semaphore()
        for nbr in (left, right):
            pl.semaphore_signal(
                barrier_sem, inc=1,
                device_id=(nbr,), device_id_type=pl.DeviceIdType.MESH,
            )
        pl.semaphore_wait(barrier_sem, 2)

        for s in range(N_DEV - 1):
            send_chunk = lax.rem(me - s + N_DEV, N_DEV)
            recv_chunk = lax.rem(me - s - 1 + N_DEV, N_DEV)
            rdma = pltpu.make_async_remote_copy(
                src_ref=out_ref.at[pl.ds(send_chunk * c, c)],
                dst_ref=comm_ref.at[s],
                send_sem=send_sems.at[s],
                recv_sem=recv_sems.at[s],
                device_id=(right,),
                device_id_type=pl.DeviceIdType.MESH,
            )
            rdma.start()
            gather_chunk(recv_chunk)
            rdma.wait()
            off = recv_chunk * c
            out_ref[pl.ds(off, c), :] = out_ref[pl.ds(off, c), :] + comm_ref[s]

        for s in range(N_DEV - 1):
            chunk = lax.rem(me + 1 - s + N_DEV, N_DEV)
            off = chunk * c
            rdma = pltpu.make_async_remote_copy(
                src_ref=out_ref.at[pl.ds(off, c)],
                dst_ref=out_ref.at[pl.ds(off, c)],
                send_sem=send_sems.at[N_DEV - 1 + s],
                recv_sem=recv_sems.at[N_DEV - 1 + s],
                device_id=(right,),
                device_id_type=pl.DeviceIdType.MESH,
            )
            rdma.start()
            rdma.wait()

    n_steps = 2 * (N_DEV - 1)
    return pl.pallas_call(
        body,
        out_shape=jax.ShapeDtypeStruct((n, d), jnp.bfloat16),
        in_specs=[
            pl.BlockSpec(memory_space=pl.ANY),
            pl.BlockSpec(memory_space=pltpu.SMEM),
            pl.BlockSpec(memory_space=pltpu.VMEM),
        ],
        out_specs=pl.BlockSpec(memory_space=pltpu.VMEM),
        scratch_shapes=[
            pltpu.VMEM((n, d), jnp.float32),
            pltpu.VMEM((N_DEV - 1, c, d), jnp.bfloat16),
            pltpu.SemaphoreType.DMA((W,)),
            pltpu.SemaphoreType.DMA((n_steps,)),
            pltpu.SemaphoreType.DMA((n_steps,)),
        ],
        compiler_params=pltpu.CompilerParams(collective_id=0),
    )(table, idx, idx2d)


# device time: 109181 ns/iter; 1.7476x vs baseline; 1.2058x over previous
import jax
import jax.numpy as jnp
from jax import lax
from jax.experimental import pallas as pl
from jax.experimental.pallas import tpu as pltpu

N_DEV = 4
W = 16


def kernel(table, idx):
    v_per, d = table.shape
    n = idx.shape[0]
    c = n // N_DEV

    my_pos = lax.axis_index("i")
    lo_x = my_pos * v_per
    pos = jnp.arange(n, dtype=jnp.int32)
    mask = (idx >= lo_x) & (idx < lo_x + v_per)
    comp_pos = jnp.argsort(jnp.where(mask, pos, n + pos)).astype(jnp.int32)
    comp_loc = jnp.clip(idx[comp_pos] - lo_x, 0, v_per - 1).astype(jnp.int32)
    num_owned = jnp.sum(mask).astype(jnp.int32)
    search_arr = jnp.where(pos < num_owned, comp_pos, n)
    bnd = jnp.searchsorted(
        search_arr, jnp.arange(0, n + 1, c, dtype=jnp.int32)
    ).astype(jnp.int32)
    idx2d = idx[:, None]

    def body(table_ref, cpos_ref, cloc_ref, bnd_ref, idx2_ref, out_ref,
             gbuf_ref, comm_ref, gather_sems, send_sems, recv_sems):
        me = lax.axis_index("i")
        left = lax.rem(me + N_DEV - 1, N_DEV)
        right = lax.rem(me + 1, N_DEV)
        lo = me * v_per

        def row_copy(j):
            return pltpu.make_async_copy(
                table_ref.at[pl.ds(cloc_ref[j], 1)],
                gbuf_ref.at[pl.ds(cpos_ref[j], 1)],
                gather_sems.at[lax.rem(j, W)],
            )

        def gather_chunk(chunk):
            start = bnd_ref[chunk]
            end = bnd_ref[chunk + 1]

            def step(j, _):
                @pl.when(j - start >= W)
                def _():
                    row_copy(j - W).wait()
                row_copy(j).start()
                return 0

            lax.fori_loop(start, end, step, 0)

            def drain(j, _):
                row_copy(j).wait()
                return 0

            lax.fori_loop(jnp.maximum(end - W, start), end, drain, 0)

            off = chunk * c
            owned = (idx2_ref[pl.ds(off, c), :] >= lo) & (
                idx2_ref[pl.ds(off, c), :] < lo + v_per
            )
            out_ref[pl.ds(off, c), :] = jnp.where(
                owned, gbuf_ref[pl.ds(off, c), :], 0.0
            ).astype(jnp.bfloat16)

        gather_chunk(me)

        barrier_sem = pltpu.get_barrier_semaphore()
        for nbr in (left, right):
            pl.semaphore_signal(
                barrier_sem, inc=1,
                device_id=(nbr,), device_id_type=pl.DeviceIdType.MESH,
            )
        pl.semaphore_wait(barrier_sem, 2)

        for s in range(N_DEV - 1):
            send_chunk = lax.rem(me - s + N_DEV, N_DEV)
            recv_chunk = lax.rem(me - s - 1 + N_DEV, N_DEV)
            rdma = pltpu.make_async_remote_copy(
                src_ref=out_ref.at[pl.ds(send_chunk * c, c)],
                dst_ref=comm_ref.at[s],
                send_sem=send_sems.at[s],
                recv_sem=recv_sems.at[s],
                device_id=(right,),
                device_id_type=pl.DeviceIdType.MESH,
            )
            rdma.start()
            gather_chunk(recv_chunk)
            rdma.wait()
            off = recv_chunk * c
            out_ref[pl.ds(off, c), :] = out_ref[pl.ds(off, c), :] + comm_ref[s]

        for s in range(N_DEV - 1):
            chunk = lax.rem(me + 1 - s + N_DEV, N_DEV)
            off = chunk * c
            rdma = pltpu.make_async_remote_copy(
                src_ref=out_ref.at[pl.ds(off, c)],
                dst_ref=out_ref.at[pl.ds(off, c)],
                send_sem=send_sems.at[N_DEV - 1 + s],
                recv_sem=recv_sems.at[N_DEV - 1 + s],
                device_id=(right,),
                device_id_type=pl.DeviceIdType.MESH,
            )
            rdma.start()
            rdma.wait()

    n_steps = 2 * (N_DEV - 1)
    return pl.pallas_call(
        body,
        out_shape=jax.ShapeDtypeStruct((n, d), jnp.bfloat16),
        in_specs=[
            pl.BlockSpec(memory_space=pl.ANY),
            pl.BlockSpec(memory_space=pltpu.SMEM),
            pl.BlockSpec(memory_space=pltpu.SMEM),
            pl.BlockSpec(memory_space=pltpu.SMEM),
            pl.BlockSpec(memory_space=pltpu.VMEM),
        ],
        out_specs=pl.BlockSpec(memory_space=pltpu.VMEM),
        scratch_shapes=[
            pltpu.VMEM((n, d), jnp.float32),
            pltpu.VMEM((N_DEV - 1, c, d), jnp.bfloat16),
            pltpu.SemaphoreType.DMA((W,)),
            pltpu.SemaphoreType.DMA((n_steps,)),
            pltpu.SemaphoreType.DMA((n_steps,)),
        ],
        compiler_params=pltpu.CompilerParams(collective_id=0),
    )(table, comp_pos, comp_loc, bnd, idx2d)


# device time: 98987 ns/iter; 1.9275x vs baseline; 1.1030x over previous
import jax
import jax.numpy as jnp
from jax import lax
from jax.experimental import pallas as pl
from jax.experimental.pallas import tpu as pltpu

N_DEV = 4
W = 16


def kernel(table, idx):
    v_per, d = table.shape
    n = idx.shape[0]
    c = n // N_DEV

    my_pos = lax.axis_index("i")
    lo_x = my_pos * v_per
    pos = jnp.arange(n, dtype=jnp.int32)
    mask = (idx >= lo_x) & (idx < lo_x + v_per)
    keys = jnp.where(mask, pos, n + pos).astype(jnp.int32)
    loc = jnp.clip(idx - lo_x, 0, v_per - 1).astype(jnp.int32)
    _, comp_pos, comp_loc = lax.sort((keys, pos, loc), num_keys=1)
    counts = jnp.sum(mask.reshape(N_DEV, c).astype(jnp.int32), axis=1)
    bnd = jnp.concatenate(
        [jnp.zeros((1,), jnp.int32), jnp.cumsum(counts).astype(jnp.int32)]
    )
    idx2d = idx[:, None]

    def body(table_ref, cpos_ref, cloc_ref, bnd_ref, idx2_ref, out_ref,
             gbuf_ref, comm_ref, gather_sems, send_sems, recv_sems):
        me = lax.axis_index("i")
        left = lax.rem(me + N_DEV - 1, N_DEV)
        right = lax.rem(me + 1, N_DEV)
        lo = me * v_per

        def row_copy(j):
            return pltpu.make_async_copy(
                table_ref.at[pl.ds(cloc_ref[j], 1)],
                gbuf_ref.at[pl.ds(cpos_ref[j], 1)],
                gather_sems.at[lax.rem(j, W)],
            )

        def gather_chunk(chunk):
            start = bnd_ref[chunk]
            end = bnd_ref[chunk + 1]

            def step(j, _):
                @pl.when(j - start >= W)
                def _():
                    row_copy(j - W).wait()
                row_copy(j).start()
                return 0

            lax.fori_loop(start, end, step, 0)

            def drain(j, _):
                row_copy(j).wait()
                return 0

            lax.fori_loop(jnp.maximum(end - W, start), end, drain, 0)

            off = chunk * c
            owned = (idx2_ref[pl.ds(off, c), :] >= lo) & (
                idx2_ref[pl.ds(off, c), :] < lo + v_per
            )
            out_ref[pl.ds(off, c), :] = jnp.where(
                owned, gbuf_ref[pl.ds(off, c), :], 0.0
            ).astype(jnp.bfloat16)

        gather_chunk(me)

        barrier_sem = pltpu.get_barrier_semaphore()
        for nbr in (left, right):
            pl.semaphore_signal(
                barrier_sem, inc=1,
                device_id=(nbr,), device_id_type=pl.DeviceIdType.MESH,
            )
        pl.semaphore_wait(barrier_sem, 2)

        for s in range(N_DEV - 1):
            send_chunk = lax.rem(me - s + N_DEV, N_DEV)
            recv_chunk = lax.rem(me - s - 1 + N_DEV, N_DEV)
            rdma = pltpu.make_async_remote_copy(
                src_ref=out_ref.at[pl.ds(send_chunk * c, c)],
                dst_ref=comm_ref.at[s],
                send_sem=send_sems.at[s],
                recv_sem=recv_sems.at[s],
                device_id=(right,),
                device_id_type=pl.DeviceIdType.MESH,
            )
            rdma.start()
            gather_chunk(recv_chunk)
            rdma.wait()
            off = recv_chunk * c
            out_ref[pl.ds(off, c), :] = out_ref[pl.ds(off, c), :] + comm_ref[s]

        for s in range(N_DEV - 1):
            chunk = lax.rem(me + 1 - s + N_DEV, N_DEV)
            off = chunk * c
            rdma = pltpu.make_async_remote_copy(
                src_ref=out_ref.at[pl.ds(off, c)],
                dst_ref=out_ref.at[pl.ds(off, c)],
                send_sem=send_sems.at[N_DEV - 1 + s],
                recv_sem=recv_sems.at[N_DEV - 1 + s],
                device_id=(right,),
                device_id_type=pl.DeviceIdType.MESH,
            )
            rdma.start()
            rdma.wait()

    n_steps = 2 * (N_DEV - 1)
    return pl.pallas_call(
        body,
        out_shape=jax.ShapeDtypeStruct((n, d), jnp.bfloat16),
        in_specs=[
            pl.BlockSpec(memory_space=pl.ANY),
            pl.BlockSpec(memory_space=pltpu.SMEM),
            pl.BlockSpec(memory_space=pltpu.SMEM),
            pl.BlockSpec(memory_space=pltpu.SMEM),
            pl.BlockSpec(memory_space=pltpu.VMEM),
        ],
        out_specs=pl.BlockSpec(memory_space=pltpu.VMEM),
        scratch_shapes=[
            pltpu.VMEM((n, d), jnp.float32),
            pltpu.VMEM((N_DEV - 1, c, d), jnp.bfloat16),
            pltpu.SemaphoreType.DMA((W,)),
            pltpu.SemaphoreType.DMA((n_steps,)),
            pltpu.SemaphoreType.DMA((n_steps,)),
        ],
        compiler_params=pltpu.CompilerParams(collective_id=0),
    )(table, comp_pos, comp_loc, bnd, idx2d)
